# baseline (speedup 1.0000x reference)
import functools

import jax
import jax.numpy as jnp
from jax import lax
from jax.experimental import pallas as pl
from jax.experimental.pallas import tpu as pltpu
from jax.experimental.pallas import tpu_sc as plsc

N = 10000
E = 320000
D = 128

NC = 2
NS = 16
NW = NC * NS
CHUNK = 128
NCHUNK = 80
IDXB = 16
EPW = NCHUNK * CHUNK
EPAD = NW * EPW
NP = 10112
RPT = NP // NS

@functools.cache
def _get_mesh():
  return plsc.VectorSubcoreMesh(core_axis_name="c", subcore_axis_name="s",
                                num_cores=NC, num_subcores=NS)


def _sc_scatter_body(s_hbm, rows_hbm, cols_hbm, zeros_hbm, out_hbm,
                     rowbuf, colbuf, g0, g1, sem0, sem1, acc):
  c = lax.axis_index("c")
  s = lax.axis_index("s")
  wid = c * NS + s
  last = s == NS - 1

  @pl.when(jnp.logical_and(c == 0, jnp.logical_not(last)))
  def _():
    pltpu.sync_copy(s_hbm.at[pl.ds(s * RPT, RPT)], acc.at[pl.ds(s * RPT, RPT)])

  @pl.when(jnp.logical_and(c == 0, last))
  def _():
    pltpu.sync_copy(s_hbm.at[pl.ds((NS - 1) * RPT, N - (NS - 1) * RPT)],
                    acc.at[pl.ds((NS - 1) * RPT, N - (NS - 1) * RPT)])

  @pl.when(c == 1)
  def _():
    pltpu.sync_copy(zeros_hbm.at[pl.ds(s * RPT, RPT)],
                    acc.at[pl.ds(s * RPT, RPT)])

  plsc.subcore_barrier()

  def blk(hb, carry):
    pltpu.sync_copy(rows_hbm.at[wid, pl.ds(hb * IDXB, IDXB)], rowbuf)
    pltpu.sync_copy(cols_hbm.at[wid, pl.ds(hb * IDXB, IDXB)], colbuf)
    pltpu.async_copy(s_hbm.at[rowbuf.at[0]], g0, sem0)

    def body(k, c2):
      j0 = 2 * k
      j1 = j0 + 1
      jn = jnp.minimum(j0 + 2, IDXB - 1)
      pltpu.make_async_copy(s_hbm.at[rowbuf.at[j0]], g0, sem0).wait()
      pltpu.async_copy(s_hbm.at[rowbuf.at[j1]], g1, sem1)
      pltpu.sync_copy(g0, acc.at[colbuf.at[j0]], add=True)
      pltpu.make_async_copy(s_hbm.at[rowbuf.at[j1]], g1, sem1).wait()
      pltpu.async_copy(s_hbm.at[rowbuf.at[jn]], g0, sem0)
      pltpu.sync_copy(g1, acc.at[colbuf.at[j1]], add=True)
      return c2

    lax.fori_loop(0, IDXB // 2, body, 0)
    pltpu.make_async_copy(s_hbm.at[rowbuf.at[IDXB - 1]], g0, sem0).wait()
    return carry

  lax.fori_loop(0, NCHUNK // IDXB, blk, 0)
  plsc.subcore_barrier()
  pltpu.sync_copy(acc.at[pl.ds(s * RPT, RPT)], out_hbm.at[c, pl.ds(s * RPT, RPT)])


def _sc_scatter(s_val, rows3d, cols3d, zerosN):
  k = pl.kernel(
      _sc_scatter_body,
      out_type=jax.ShapeDtypeStruct((NC, NP, D), jnp.float32),
      mesh=_get_mesh(),
      scratch_types=[
          pltpu.VMEM((IDXB, CHUNK), jnp.int32),
          pltpu.VMEM((IDXB, CHUNK), jnp.int32),
          pltpu.VMEM((CHUNK, D), jnp.float32),
          pltpu.VMEM((CHUNK, D), jnp.float32),
          pltpu.SemaphoreType.DMA,
          pltpu.SemaphoreType.DMA,
          pltpu.VMEM_SHARED((NP, D), jnp.float32),
      ],
  )
  return k(s_val, rows3d, cols3d, zerosN)


def _sc_deg_body(ones_hbm, cols_hbm, zeros_hbm, out_hbm, colbuf, onesbuf, acc):
  c = lax.axis_index("c")
  s = lax.axis_index("s")
  wid = c * NS + s
  pltpu.sync_copy(ones_hbm.at[pl.ds(0, CHUNK)], onesbuf)
  last = s == NS - 1

  @pl.when(jnp.logical_and(c == 0, jnp.logical_not(last)))
  def _():
    pltpu.sync_copy(ones_hbm.at[pl.ds(s * RPT, RPT)], acc.at[pl.ds(s * RPT, RPT)])

  @pl.when(jnp.logical_and(c == 0, last))
  def _():
    pltpu.sync_copy(ones_hbm.at[pl.ds((NS - 1) * RPT, N - (NS - 1) * RPT)],
                    acc.at[pl.ds((NS - 1) * RPT, N - (NS - 1) * RPT)])

  @pl.when(c == 1)
  def _():
    pltpu.sync_copy(zeros_hbm.at[pl.ds(s * RPT, RPT)],
                    acc.at[pl.ds(s * RPT, RPT)])

  plsc.subcore_barrier()

  def blk(hb, carry):
    pltpu.sync_copy(cols_hbm.at[wid, pl.ds(hb * IDXB, IDXB)], colbuf)

    def body(j, c2):
      pltpu.sync_copy(onesbuf, acc.at[colbuf.at[j]], add=True)
      return c2

    lax.fori_loop(0, IDXB, body, 0)
    return carry

  lax.fori_loop(0, NCHUNK // IDXB, blk, 0)
  plsc.subcore_barrier()
  pltpu.sync_copy(acc.at[pl.ds(s * RPT, RPT)], out_hbm.at[c, pl.ds(s * RPT, RPT)])


def _sc_deg(onesN, cols3d, zerosN):
  k = pl.kernel(
      _sc_deg_body,
      out_type=jax.ShapeDtypeStruct((NC, NP, D), jnp.float32),
      mesh=_get_mesh(),
      scratch_types=[
          pltpu.VMEM((IDXB, CHUNK), jnp.int32),
          pltpu.VMEM((CHUNK, D), jnp.float32),
          pltpu.VMEM_SHARED((NP, D), jnp.float32),
      ],
  )
  return k(onesN, cols3d, zerosN)


R = 1000
G = N // R

_DOT = dict(precision=lax.Precision.HIGHEST, preferred_element_type=jnp.float32)


def _ln(h, g, b):
  mu = jnp.mean(h, axis=-1, keepdims=True)
  d = h - mu
  var = jnp.mean(d * d, axis=-1, keepdims=True)
  return d * lax.rsqrt(var + 1e-5) * g + b


def _dinv(dd):
  return lax.rsqrt(dd[0, :, 0:1] + dd[1, :, 0:1])


def _gelu(x):
  return 0.5 * x * (1.0 + lax.erf(x * 0.7071067811865476))


def _tc1_body(x_ref, wp_ref, bp_ref, gp_ref, bep_ref, w1_ref, dd_ref,
              h0_ref, s1_ref):
  h = lax.dot_general(x_ref[...], wp_ref[...], (((1,), (1,)), ((), ())), **_DOT)
  h = _ln(h + bp_ref[...], gp_ref[...], bep_ref[...])
  h0_ref[...] = h
  s1_ref[...] = _dinv(dd_ref) * lax.dot_general(
      h, w1_ref[...], (((1,), (1,)), ((), ())), **_DOT)


def _tc_mid_body(h_ref, p_ref, dd_ref, b_ref, g_ref, be_ref, wn_ref,
                 hn_ref, sn_ref):
  dinv = _dinv(dd_ref)
  m = _gelu(dinv * (p_ref[0] + p_ref[1]) + b_ref[...])
  hn = _ln(h_ref[...] + m, g_ref[...], be_ref[...])
  hn_ref[...] = hn
  sn_ref[...] = dinv * lax.dot_general(
      hn, wn_ref[...], (((1,), (1,)), ((), ())), **_DOT)


def _tc_fin_body(h_ref, p_ref, dd_ref, b_ref, g_ref, be_ref, hn_ref):
  dinv = _dinv(dd_ref)
  m = _gelu(dinv * (p_ref[0] + p_ref[1]) + b_ref[...])
  hn_ref[...] = _ln(h_ref[...] + m, g_ref[...], be_ref[...])


_ROW = pl.BlockSpec((R, D), lambda i: (i, 0))
_W = pl.BlockSpec((D, D), lambda i: (0, 0))
_VEC = pl.BlockSpec((1, D), lambda i: (0, 0))
_DD = pl.BlockSpec((NC, R, D), lambda i: (0, i, 0))
_P = pl.BlockSpec((NC, R, D), lambda i: (0, i, 0))
_OUT2 = [jax.ShapeDtypeStruct((N, D), jnp.float32)] * 2
_OUT1 = jax.ShapeDtypeStruct((N, D), jnp.float32)


def _tc1(x, wp, bp, gp, bep, w1, dd):
  return pl.pallas_call(
      _tc1_body, grid=(G,),
      in_specs=[_ROW, _W, _VEC, _VEC, _VEC, _W, _DD],
      out_specs=[_ROW, _ROW], out_shape=_OUT2,
  )(x, wp, bp, gp, bep, w1, dd)


def _tc_mid(h, p, dd, b, g, be, wn):
  return pl.pallas_call(
      _tc_mid_body, grid=(G,),
      in_specs=[_ROW, _P, _DD, _VEC, _VEC, _VEC, _W],
      out_specs=[_ROW, _ROW], out_shape=_OUT2,
  )(h, p, dd, b, g, be, wn)


def _tc_fin(h, p, dd, b, g, be):
  return pl.pallas_call(
      _tc_fin_body, grid=(G,),
      in_specs=[_ROW, _P, _DD, _VEC, _VEC, _VEC],
      out_specs=_ROW, out_shape=_OUT1,
  )(h, p, dd, b, g, be)


def kernel(x, edge_index, W_proj, b_proj, g_proj, be_proj,
           W1, b1, g1, be1, W2, b2, g2, be2, W3, b3, g3, be3):
  pad = EPAD - E
  rows = jnp.concatenate([edge_index[0], jnp.zeros((pad,), jnp.int32)])
  cols = jnp.concatenate([edge_index[1], jnp.full((pad,), N, jnp.int32)])
  rows3d = rows.reshape(NW, NCHUNK, CHUNK)
  cols3d = cols.reshape(NW, NCHUNK, CHUNK)
  zerosN = jnp.zeros((NP, D), jnp.float32)
  onesN = jnp.ones((N, D), jnp.float32)

  dd = _sc_deg(onesN, cols3d, zerosN)

  r2 = lambda v: v.reshape(1, D)
  h0, s1 = _tc1(x, W_proj, r2(b_proj), r2(g_proj), r2(be_proj), W1, dd)
  p1 = _sc_scatter(s1, rows3d, cols3d, zerosN)
  h1, s2 = _tc_mid(h0, p1, dd, r2(b1), r2(g1), r2(be1), W2)
  p2 = _sc_scatter(s2, rows3d, cols3d, zerosN)
  h2, s3 = _tc_mid(h1, p2, dd, r2(b2), r2(g2), r2(be2), W3)
  p3 = _sc_scatter(s3, rows3d, cols3d, zerosN)
  return _tc_fin(h2, p3, dd, r2(b3), r2(g3), r2(be3))


# speedup vs baseline: 1.0317x; 1.0317x over previous
import functools

import jax
import jax.numpy as jnp
from jax import lax
from jax.experimental import pallas as pl
from jax.experimental.pallas import tpu as pltpu
from jax.experimental.pallas import tpu_sc as plsc

N = 10000
E = 320000
D = 128

NC = 2
NS = 16
NW = NC * NS
CHUNK = 128
NCHUNK = 80
IDXB = 40
EPW = NCHUNK * CHUNK
EPAD = NW * EPW
NP = 10112
RPT = NP // NS

@functools.cache
def _get_mesh():
  return plsc.VectorSubcoreMesh(core_axis_name="c", subcore_axis_name="s",
                                num_cores=NC, num_subcores=NS)


def _sc_scatter_body(s_hbm, rows_hbm, cols_hbm, zeros_hbm, out_hbm,
                     rowbuf, colbuf, g0, g1, sem0, sem1, acc):
  c = lax.axis_index("c")
  s = lax.axis_index("s")
  wid = c * NS + s
  last = s == NS - 1

  @pl.when(jnp.logical_and(c == 0, jnp.logical_not(last)))
  def _():
    pltpu.sync_copy(s_hbm.at[pl.ds(s * RPT, RPT)], acc.at[pl.ds(s * RPT, RPT)])

  @pl.when(jnp.logical_and(c == 0, last))
  def _():
    pltpu.sync_copy(s_hbm.at[pl.ds((NS - 1) * RPT, N - (NS - 1) * RPT)],
                    acc.at[pl.ds((NS - 1) * RPT, N - (NS - 1) * RPT)])

  @pl.when(c == 1)
  def _():
    pltpu.sync_copy(zeros_hbm.at[pl.ds(s * RPT, RPT)],
                    acc.at[pl.ds(s * RPT, RPT)])

  plsc.subcore_barrier()

  def blk(hb, carry):
    pltpu.sync_copy(rows_hbm.at[wid, pl.ds(hb * IDXB, IDXB)], rowbuf)
    pltpu.sync_copy(cols_hbm.at[wid, pl.ds(hb * IDXB, IDXB)], colbuf)
    pltpu.async_copy(s_hbm.at[rowbuf.at[0]], g0, sem0)

    def body(k, c2):
      j0 = 2 * k
      j1 = j0 + 1
      jn = jnp.minimum(j0 + 2, IDXB - 1)
      pltpu.make_async_copy(s_hbm.at[rowbuf.at[j0]], g0, sem0).wait()
      pltpu.async_copy(s_hbm.at[rowbuf.at[j1]], g1, sem1)
      pltpu.sync_copy(g0, acc.at[colbuf.at[j0]], add=True)
      pltpu.make_async_copy(s_hbm.at[rowbuf.at[j1]], g1, sem1).wait()
      pltpu.async_copy(s_hbm.at[rowbuf.at[jn]], g0, sem0)
      pltpu.sync_copy(g1, acc.at[colbuf.at[j1]], add=True)
      return c2

    lax.fori_loop(0, IDXB // 2, body, 0)
    pltpu.make_async_copy(s_hbm.at[rowbuf.at[IDXB - 1]], g0, sem0).wait()
    return carry

  lax.fori_loop(0, NCHUNK // IDXB, blk, 0)
  plsc.subcore_barrier()
  pltpu.sync_copy(acc.at[pl.ds(s * RPT, RPT)], out_hbm.at[c, pl.ds(s * RPT, RPT)])


def _sc_scatter(s_val, rows3d, cols3d, zerosN):
  k = pl.kernel(
      _sc_scatter_body,
      out_type=jax.ShapeDtypeStruct((NC, NP, D), jnp.float32),
      mesh=_get_mesh(),
      scratch_types=[
          pltpu.VMEM((IDXB, CHUNK), jnp.int32),
          pltpu.VMEM((IDXB, CHUNK), jnp.int32),
          pltpu.VMEM((CHUNK, D), jnp.float32),
          pltpu.VMEM((CHUNK, D), jnp.float32),
          pltpu.SemaphoreType.DMA,
          pltpu.SemaphoreType.DMA,
          pltpu.VMEM_SHARED((NP, D), jnp.float32),
      ],
  )
  return k(s_val, rows3d, cols3d, zerosN)


def _sc_deg_body(ones_hbm, cols_hbm, zeros_hbm, out_hbm, colbuf, onesbuf, sem0,
                 acc):
  c = lax.axis_index("c")
  s = lax.axis_index("s")
  wid = c * NS + s
  pltpu.sync_copy(ones_hbm.at[pl.ds(0, CHUNK)], onesbuf)
  last = s == NS - 1

  @pl.when(jnp.logical_and(c == 0, jnp.logical_not(last)))
  def _():
    pltpu.sync_copy(ones_hbm.at[pl.ds(s * RPT, RPT)], acc.at[pl.ds(s * RPT, RPT)])

  @pl.when(jnp.logical_and(c == 0, last))
  def _():
    pltpu.sync_copy(ones_hbm.at[pl.ds((NS - 1) * RPT, N - (NS - 1) * RPT)],
                    acc.at[pl.ds((NS - 1) * RPT, N - (NS - 1) * RPT)])

  @pl.when(c == 1)
  def _():
    pltpu.sync_copy(zeros_hbm.at[pl.ds(s * RPT, RPT)],
                    acc.at[pl.ds(s * RPT, RPT)])

  plsc.subcore_barrier()

  def blk(hb, carry):
    pltpu.sync_copy(cols_hbm.at[wid, pl.ds(hb * IDXB, IDXB)], colbuf)

    def body(j, c2):
      pltpu.async_copy(onesbuf, acc.at[colbuf.at[j]], sem0, add=True)
      return c2

    lax.fori_loop(0, IDXB, body, 0)

    def drain(j, c2):
      pltpu.make_async_copy(onesbuf, acc.at[colbuf.at[j]], sem0).wait()
      return c2

    lax.fori_loop(0, IDXB, drain, 0)
    return carry

  lax.fori_loop(0, NCHUNK // IDXB, blk, 0)
  plsc.subcore_barrier()
  pltpu.sync_copy(acc.at[pl.ds(s * RPT, RPT)], out_hbm.at[c, pl.ds(s * RPT, RPT)])


def _sc_deg(onesN, cols3d, zerosN):
  k = pl.kernel(
      _sc_deg_body,
      out_type=jax.ShapeDtypeStruct((NC, NP, D), jnp.float32),
      mesh=_get_mesh(),
      scratch_types=[
          pltpu.VMEM((IDXB, CHUNK), jnp.int32),
          pltpu.VMEM((CHUNK, D), jnp.float32),
          pltpu.SemaphoreType.DMA,
          pltpu.VMEM_SHARED((NP, D), jnp.float32),
      ],
  )
  return k(onesN, cols3d, zerosN)


R = 1000
G = N // R

_DOT = dict(precision=lax.Precision.HIGHEST, preferred_element_type=jnp.float32)


def _ln(h, g, b):
  mu = jnp.mean(h, axis=-1, keepdims=True)
  d = h - mu
  var = jnp.mean(d * d, axis=-1, keepdims=True)
  return d * lax.rsqrt(var + 1e-5) * g + b


def _dinv(dd):
  return lax.rsqrt(dd[0, :, 0:1] + dd[1, :, 0:1])


def _gelu(x):
  return 0.5 * x * (1.0 + lax.erf(x * 0.7071067811865476))


def _tc1_body(x_ref, wp_ref, bp_ref, gp_ref, bep_ref, w1_ref, dd_ref,
              h0_ref, s1_ref):
  h = lax.dot_general(x_ref[...], wp_ref[...], (((1,), (1,)), ((), ())), **_DOT)
  h = _ln(h + bp_ref[...], gp_ref[...], bep_ref[...])
  h0_ref[...] = h
  s1_ref[...] = _dinv(dd_ref) * lax.dot_general(
      h, w1_ref[...], (((1,), (1,)), ((), ())), **_DOT)


def _tc_mid_body(h_ref, p_ref, dd_ref, b_ref, g_ref, be_ref, wn_ref,
                 hn_ref, sn_ref):
  dinv = _dinv(dd_ref)
  m = _gelu(dinv * (p_ref[0] + p_ref[1]) + b_ref[...])
  hn = _ln(h_ref[...] + m, g_ref[...], be_ref[...])
  hn_ref[...] = hn
  sn_ref[...] = dinv * lax.dot_general(
      hn, wn_ref[...], (((1,), (1,)), ((), ())), **_DOT)


def _tc_fin_body(h_ref, p_ref, dd_ref, b_ref, g_ref, be_ref, hn_ref):
  dinv = _dinv(dd_ref)
  m = _gelu(dinv * (p_ref[0] + p_ref[1]) + b_ref[...])
  hn_ref[...] = _ln(h_ref[...] + m, g_ref[...], be_ref[...])


_ROW = pl.BlockSpec((R, D), lambda i: (i, 0))
_W = pl.BlockSpec((D, D), lambda i: (0, 0))
_VEC = pl.BlockSpec((1, D), lambda i: (0, 0))
_DD = pl.BlockSpec((NC, R, D), lambda i: (0, i, 0))
_P = pl.BlockSpec((NC, R, D), lambda i: (0, i, 0))
_OUT2 = [jax.ShapeDtypeStruct((N, D), jnp.float32)] * 2
_OUT1 = jax.ShapeDtypeStruct((N, D), jnp.float32)


def _tc1(x, wp, bp, gp, bep, w1, dd):
  return pl.pallas_call(
      _tc1_body, grid=(G,),
      in_specs=[_ROW, _W, _VEC, _VEC, _VEC, _W, _DD],
      out_specs=[_ROW, _ROW], out_shape=_OUT2,
  )(x, wp, bp, gp, bep, w1, dd)


def _tc_mid(h, p, dd, b, g, be, wn):
  return pl.pallas_call(
      _tc_mid_body, grid=(G,),
      in_specs=[_ROW, _P, _DD, _VEC, _VEC, _VEC, _W],
      out_specs=[_ROW, _ROW], out_shape=_OUT2,
  )(h, p, dd, b, g, be, wn)


def _tc_fin(h, p, dd, b, g, be):
  return pl.pallas_call(
      _tc_fin_body, grid=(G,),
      in_specs=[_ROW, _P, _DD, _VEC, _VEC, _VEC],
      out_specs=_ROW, out_shape=_OUT1,
  )(h, p, dd, b, g, be)


def kernel(x, edge_index, W_proj, b_proj, g_proj, be_proj,
           W1, b1, g1, be1, W2, b2, g2, be2, W3, b3, g3, be3):
  pad = EPAD - E
  rows = jnp.concatenate([edge_index[0], jnp.zeros((pad,), jnp.int32)])
  cols = jnp.concatenate([edge_index[1], jnp.full((pad,), N, jnp.int32)])
  rows3d = rows.reshape(NW, NCHUNK, CHUNK)
  cols3d = cols.reshape(NW, NCHUNK, CHUNK)
  zerosN = jnp.zeros((NP, D), jnp.float32)
  onesN = jnp.ones((N, D), jnp.float32)

  dd = _sc_deg(onesN, cols3d, zerosN)

  r2 = lambda v: v.reshape(1, D)
  h0, s1 = _tc1(x, W_proj, r2(b_proj), r2(g_proj), r2(be_proj), W1, dd)
  p1 = _sc_scatter(s1, rows3d, cols3d, zerosN)
  h1, s2 = _tc_mid(h0, p1, dd, r2(b1), r2(g1), r2(be1), W2)
  p2 = _sc_scatter(s2, rows3d, cols3d, zerosN)
  h2, s3 = _tc_mid(h1, p2, dd, r2(b2), r2(g2), r2(be2), W3)
  p3 = _sc_scatter(s3, rows3d, cols3d, zerosN)
  return _tc_fin(h2, p3, dd, r2(b3), r2(g3), r2(be3))


# speedup vs baseline: 1.6732x; 1.6217x over previous
import functools

import jax
import jax.numpy as jnp
from jax import lax
from jax.experimental import pallas as pl
from jax.experimental.pallas import tpu as pltpu
from jax.experimental.pallas import tpu_sc as plsc

N = 10000
E = 320000
D = 128

NC = 2
NS = 16
NW = NC * NS
CHUNK = 105
NCHUNK = 96
IDXB = 24
NBLK = NCHUNK // IDXB
EPW = NCHUNK * CHUNK
EPAD = NW * EPW
NP = 10008
RPT = 632
RL_OUT = NP - 15 * RPT
RL_S = N - 15 * RPT

@functools.cache
def _get_mesh():
  return plsc.VectorSubcoreMesh(core_axis_name="c", subcore_axis_name="s",
                                num_cores=NC, num_subcores=NS)


def _init_acc(c, s, init_hbm, zeros_hbm, acc):
  last = s == NS - 1

  @pl.when(jnp.logical_and(c == 0, jnp.logical_not(last)))
  def _():
    pltpu.sync_copy(init_hbm.at[pl.ds(s * RPT, RPT)],
                    acc.at[pl.ds(s * RPT, RPT)])

  @pl.when(jnp.logical_and(c == 0, last))
  def _():
    pltpu.sync_copy(init_hbm.at[pl.ds(15 * RPT, RL_S)],
                    acc.at[pl.ds(15 * RPT, RL_S)])

  @pl.when(jnp.logical_and(c == 1, jnp.logical_not(last)))
  def _():
    pltpu.sync_copy(zeros_hbm.at[pl.ds(s * RPT, RPT)],
                    acc.at[pl.ds(s * RPT, RPT)])

  @pl.when(jnp.logical_and(c == 1, last))
  def _():
    pltpu.sync_copy(zeros_hbm.at[pl.ds(15 * RPT, RL_OUT)],
                    acc.at[pl.ds(15 * RPT, RL_OUT)])


def _writeout(c, s, acc, out_hbm):
  last = s == NS - 1

  @pl.when(jnp.logical_not(last))
  def _():
    pltpu.sync_copy(acc.at[pl.ds(s * RPT, RPT)],
                    out_hbm.at[c, pl.ds(s * RPT, RPT)])

  @pl.when(last)
  def _():
    pltpu.sync_copy(acc.at[pl.ds(15 * RPT, RL_OUT)],
                    out_hbm.at[c, pl.ds(15 * RPT, RL_OUT)])


def _sc_scatter_body(s_hbm, rows_hbm, cols_hbm, zeros_hbm, out_hbm,
                     rowA, colA, rowB, colB, g0, g1,
                     isA, isB, gs0, gs1, ss0, ss1, acc):
  c = lax.axis_index("c")
  s = lax.axis_index("s")
  wid = c * NS + s
  pltpu.async_copy(rows_hbm.at[wid, pl.ds(0, IDXB)], rowA, isA)
  pltpu.async_copy(cols_hbm.at[wid, pl.ds(0, IDXB)], colA, isA)
  pltpu.async_copy(rows_hbm.at[wid, pl.ds(IDXB, IDXB)], rowB, isB)
  pltpu.async_copy(cols_hbm.at[wid, pl.ds(IDXB, IDXB)], colB, isB)
  _init_acc(c, s, s_hbm, zeros_hbm, acc)
  plsc.subcore_barrier()

  def process(rb, cb, isem):
    pltpu.make_async_copy(rows_hbm.at[wid, pl.ds(0, IDXB)], rb, isem).wait()
    pltpu.make_async_copy(cols_hbm.at[wid, pl.ds(0, IDXB)], cb, isem).wait()
    pltpu.async_copy(s_hbm.at[rb.at[0]], g0, gs0)
    pltpu.async_copy(s_hbm.at[rb.at[1]], g1, gs1)

    def body(k, c2):
      j0 = 2 * k
      j1 = j0 + 1
      jn0 = jnp.minimum(j0 + 2, IDXB - 1)
      jn1 = jnp.minimum(j0 + 3, IDXB - 1)
      pltpu.make_async_copy(s_hbm.at[rb.at[j0]], g0, gs0).wait()
      pltpu.async_copy(g0, acc.at[cb.at[j0]], ss0, add=True)
      pltpu.make_async_copy(s_hbm.at[rb.at[j1]], g1, gs1).wait()
      pltpu.async_copy(g1, acc.at[cb.at[j1]], ss1, add=True)
      pltpu.make_async_copy(g0, acc.at[cb.at[j0]], ss0).wait()
      pltpu.async_copy(s_hbm.at[rb.at[jn0]], g0, gs0)
      pltpu.make_async_copy(g1, acc.at[cb.at[j1]], ss1).wait()
      pltpu.async_copy(s_hbm.at[rb.at[jn1]], g1, gs1)
      return c2

    lax.fori_loop(0, IDXB // 2, body, 0)
    pltpu.make_async_copy(s_hbm.at[rb.at[IDXB - 1]], g0, gs0).wait()
    pltpu.make_async_copy(s_hbm.at[rb.at[IDXB - 1]], g1, gs1).wait()

  def prefetch(rb, cb, isem, blk):
    pltpu.async_copy(rows_hbm.at[wid, pl.ds(blk * IDXB, IDXB)], rb, isem)
    pltpu.async_copy(cols_hbm.at[wid, pl.ds(blk * IDXB, IDXB)], cb, isem)

  def hbody(h, carry):
    process(rowA, colA, isA)
    prefetch(rowA, colA, isA, jnp.minimum(2 * h + 2, NBLK - 1))
    process(rowB, colB, isB)
    prefetch(rowB, colB, isB, jnp.minimum(2 * h + 3, NBLK - 1))
    return carry

  lax.fori_loop(0, NBLK // 2, hbody, 0)
  pltpu.make_async_copy(rows_hbm.at[wid, pl.ds(0, IDXB)], rowA, isA).wait()
  pltpu.make_async_copy(cols_hbm.at[wid, pl.ds(0, IDXB)], colA, isA).wait()
  pltpu.make_async_copy(rows_hbm.at[wid, pl.ds(0, IDXB)], rowB, isB).wait()
  pltpu.make_async_copy(cols_hbm.at[wid, pl.ds(0, IDXB)], colB, isB).wait()
  plsc.subcore_barrier()
  _writeout(c, s, acc, out_hbm)


def _sc_scatter(s_val, rows3d, cols3d, zerosN):
  k = pl.kernel(
      _sc_scatter_body,
      out_type=jax.ShapeDtypeStruct((NC, NP, D), jnp.float32),
      mesh=_get_mesh(),
      scratch_types=[
          pltpu.VMEM((IDXB, CHUNK), jnp.int32),
          pltpu.VMEM((IDXB, CHUNK), jnp.int32),
          pltpu.VMEM((IDXB, CHUNK), jnp.int32),
          pltpu.VMEM((IDXB, CHUNK), jnp.int32),
          pltpu.VMEM((CHUNK, D), jnp.float32),
          pltpu.VMEM((CHUNK, D), jnp.float32),
          pltpu.SemaphoreType.DMA,
          pltpu.SemaphoreType.DMA,
          pltpu.SemaphoreType.DMA,
          pltpu.SemaphoreType.DMA,
          pltpu.SemaphoreType.DMA,
          pltpu.SemaphoreType.DMA,
          pltpu.VMEM_SHARED((NP, D), jnp.float32),
      ],
  )
  return k(s_val, rows3d, cols3d, zerosN)


def _sc_deg_body(ones_hbm, cols_hbm, zeros_hbm, out_hbm, colbuf, onesbuf, sem0,
                 acc):
  c = lax.axis_index("c")
  s = lax.axis_index("s")
  wid = c * NS + s
  pltpu.sync_copy(ones_hbm.at[pl.ds(0, 112)], onesbuf)
  _init_acc(c, s, ones_hbm, zeros_hbm, acc)
  pltpu.sync_copy(cols_hbm.at[wid], colbuf)
  plsc.subcore_barrier()

  def body(j, c2):
    pltpu.async_copy(onesbuf.at[pl.ds(0, CHUNK)], acc.at[colbuf.at[j]], sem0,
                     add=True)
    return c2

  lax.fori_loop(0, NCHUNK, body, 0)

  def drain(j, c2):
    pltpu.make_async_copy(onesbuf.at[pl.ds(0, CHUNK)], acc.at[colbuf.at[j]],
                          sem0).wait()
    return c2

  lax.fori_loop(0, NCHUNK, drain, 0)
  plsc.subcore_barrier()
  _writeout(c, s, acc, out_hbm)


def _sc_deg(onesN, cols3d, zerosN):
  k = pl.kernel(
      _sc_deg_body,
      out_type=jax.ShapeDtypeStruct((NC, NP, D), jnp.float32),
      mesh=_get_mesh(),
      scratch_types=[
          pltpu.VMEM((NCHUNK, CHUNK), jnp.int32),
          pltpu.VMEM((112, D), jnp.float32),
          pltpu.SemaphoreType.DMA,
          pltpu.VMEM_SHARED((NP, D), jnp.float32),
      ],
  )
  return k(onesN, cols3d, zerosN)


R = 1000
G = N // R

_DOT = dict(precision=lax.Precision.HIGHEST, preferred_element_type=jnp.float32)


def _ln(h, g, b):
  mu = jnp.mean(h, axis=-1, keepdims=True)
  d = h - mu
  var = jnp.mean(d * d, axis=-1, keepdims=True)
  return d * lax.rsqrt(var + 1e-5) * g + b


def _dinv(dd):
  return lax.rsqrt(dd[0, :, 0:1] + dd[1, :, 0:1])


def _gelu(x):
  return 0.5 * x * (1.0 + lax.erf(x * 0.7071067811865476))


def _tc1_body(x_ref, wp_ref, bp_ref, gp_ref, bep_ref, w1_ref, dd_ref,
              h0_ref, s1_ref):
  h = lax.dot_general(x_ref[...], wp_ref[...], (((1,), (1,)), ((), ())), **_DOT)
  h = _ln(h + bp_ref[...], gp_ref[...], bep_ref[...])
  h0_ref[...] = h
  s1_ref[...] = _dinv(dd_ref) * lax.dot_general(
      h, w1_ref[...], (((1,), (1,)), ((), ())), **_DOT)


def _tc_mid_body(h_ref, p_ref, dd_ref, b_ref, g_ref, be_ref, wn_ref,
                 hn_ref, sn_ref):
  dinv = _dinv(dd_ref)
  m = _gelu(dinv * (p_ref[0] + p_ref[1]) + b_ref[...])
  hn = _ln(h_ref[...] + m, g_ref[...], be_ref[...])
  hn_ref[...] = hn
  sn_ref[...] = dinv * lax.dot_general(
      hn, wn_ref[...], (((1,), (1,)), ((), ())), **_DOT)


def _tc_fin_body(h_ref, p_ref, dd_ref, b_ref, g_ref, be_ref, hn_ref):
  dinv = _dinv(dd_ref)
  m = _gelu(dinv * (p_ref[0] + p_ref[1]) + b_ref[...])
  hn_ref[...] = _ln(h_ref[...] + m, g_ref[...], be_ref[...])


_ROW = pl.BlockSpec((R, D), lambda i: (i, 0))
_W = pl.BlockSpec((D, D), lambda i: (0, 0))
_VEC = pl.BlockSpec((1, D), lambda i: (0, 0))
_DD = pl.BlockSpec((NC, R, D), lambda i: (0, i, 0))
_P = pl.BlockSpec((NC, R, D), lambda i: (0, i, 0))
_OUT2 = [jax.ShapeDtypeStruct((N, D), jnp.float32)] * 2
_OUT1 = jax.ShapeDtypeStruct((N, D), jnp.float32)


def _tc1(x, wp, bp, gp, bep, w1, dd):
  return pl.pallas_call(
      _tc1_body, grid=(G,),
      in_specs=[_ROW, _W, _VEC, _VEC, _VEC, _W, _DD],
      out_specs=[_ROW, _ROW], out_shape=_OUT2,
  )(x, wp, bp, gp, bep, w1, dd)


def _tc_mid(h, p, dd, b, g, be, wn):
  return pl.pallas_call(
      _tc_mid_body, grid=(G,),
      in_specs=[_ROW, _P, _DD, _VEC, _VEC, _VEC, _W],
      out_specs=[_ROW, _ROW], out_shape=_OUT2,
  )(h, p, dd, b, g, be, wn)


def _tc_fin(h, p, dd, b, g, be):
  return pl.pallas_call(
      _tc_fin_body, grid=(G,),
      in_specs=[_ROW, _P, _DD, _VEC, _VEC, _VEC],
      out_specs=_ROW, out_shape=_OUT1,
  )(h, p, dd, b, g, be)


def kernel(x, edge_index, W_proj, b_proj, g_proj, be_proj,
           W1, b1, g1, be1, W2, b2, g2, be2, W3, b3, g3, be3):
  pad = EPAD - E
  rows = jnp.concatenate([edge_index[0], jnp.zeros((pad,), jnp.int32)])
  cols = jnp.concatenate([edge_index[1], jnp.full((pad,), N, jnp.int32)])
  rows3d = rows.reshape(NW, NCHUNK, CHUNK)
  cols3d = cols.reshape(NW, NCHUNK, CHUNK)
  zerosN = jnp.zeros((NP, D), jnp.float32)
  onesN = jnp.ones((N, D), jnp.float32)

  dd = _sc_deg(onesN, cols3d, zerosN)

  r2 = lambda v: v.reshape(1, D)
  h0, s1 = _tc1(x, W_proj, r2(b_proj), r2(g_proj), r2(be_proj), W1, dd)
  p1 = _sc_scatter(s1, rows3d, cols3d, zerosN)
  h1, s2 = _tc_mid(h0, p1, dd, r2(b1), r2(g1), r2(be1), W2)
  p2 = _sc_scatter(s2, rows3d, cols3d, zerosN)
  h2, s3 = _tc_mid(h1, p2, dd, r2(b2), r2(g2), r2(be2), W3)
  p3 = _sc_scatter(s3, rows3d, cols3d, zerosN)
  return _tc_fin(h2, p3, dd, r2(b3), r2(g3), r2(be3))


# speedup vs baseline: 1.9258x; 1.1510x over previous
import functools

import jax
import jax.numpy as jnp
from jax import lax
from jax.experimental import pallas as pl
from jax.experimental.pallas import tpu as pltpu
from jax.experimental.pallas import tpu_sc as plsc

N = 10000
E = 320000
D = 128

NC = 2
NS = 16
NW = NC * NS
CHUNK = 105
IDXB = 24
NCH0 = 144
NCH1 = 48
TOTC = NS * (NCH0 + NCH1) + 96
NPROC = NS * (NCH0 + NCH1)
EPAD = TOTC * CHUNK
NP = 10008
RPT = 632
RL_OUT = NP - 15 * RPT
RL_S = N - 15 * RPT

@functools.cache
def _get_mesh():
  return plsc.VectorSubcoreMesh(core_axis_name="c", subcore_axis_name="s",
                                num_cores=NC, num_subcores=NS)


def _init_acc(c, s, init_hbm, zeros_hbm, acc):
  last = s == NS - 1

  @pl.when(jnp.logical_and(c == 0, jnp.logical_not(last)))
  def _():
    pltpu.sync_copy(init_hbm.at[pl.ds(s * RPT, RPT)],
                    acc.at[pl.ds(s * RPT, RPT)])

  @pl.when(jnp.logical_and(c == 0, last))
  def _():
    pltpu.sync_copy(init_hbm.at[pl.ds(15 * RPT, RL_S)],
                    acc.at[pl.ds(15 * RPT, RL_S)])

  @pl.when(jnp.logical_and(c == 1, jnp.logical_not(last)))
  def _():
    pltpu.sync_copy(zeros_hbm.at[pl.ds(s * RPT, RPT)],
                    acc.at[pl.ds(s * RPT, RPT)])

  @pl.when(jnp.logical_and(c == 1, last))
  def _():
    pltpu.sync_copy(zeros_hbm.at[pl.ds(15 * RPT, RL_OUT)],
                    acc.at[pl.ds(15 * RPT, RL_OUT)])


def _writeout(c, s, acc, out_hbm):
  last = s == NS - 1

  @pl.when(jnp.logical_not(last))
  def _():
    pltpu.sync_copy(acc.at[pl.ds(s * RPT, RPT)],
                    out_hbm.at[c, pl.ds(s * RPT, RPT)])

  @pl.when(last)
  def _():
    pltpu.sync_copy(acc.at[pl.ds(15 * RPT, RL_OUT)],
                    out_hbm.at[c, pl.ds(15 * RPT, RL_OUT)])


def _sc_scatter_body(s_hbm, rows_hbm, cols_hbm, zeros_hbm, out_hbm,
                     rowA, colA, rowB, colB, g0, g1,
                     isA, isB, gs0, gs1, ss0, ss1, acc):
  c = lax.axis_index("c")
  s = lax.axis_index("s")
  cbase = jnp.where(c == 0, s * NCH0, NS * NCH0 + s * NCH1)
  nblk = jnp.where(c == 0, NCH0 // IDXB, NCH1 // IDXB)
  npairs = jnp.where(c == 0, NCH0 // (2 * IDXB), NCH1 // (2 * IDXB))
  pltpu.async_copy(rows_hbm.at[pl.ds(cbase, IDXB)], rowA, isA)
  pltpu.async_copy(cols_hbm.at[pl.ds(cbase, IDXB)], colA, isA)
  pltpu.async_copy(rows_hbm.at[pl.ds(cbase + IDXB, IDXB)], rowB, isB)
  pltpu.async_copy(cols_hbm.at[pl.ds(cbase + IDXB, IDXB)], colB, isB)
  _init_acc(c, s, s_hbm, zeros_hbm, acc)
  plsc.subcore_barrier()

  def process(rb, cb, isem):
    pltpu.make_async_copy(rows_hbm.at[pl.ds(0, IDXB)], rb, isem).wait()
    pltpu.make_async_copy(cols_hbm.at[pl.ds(0, IDXB)], cb, isem).wait()
    pltpu.async_copy(s_hbm.at[rb.at[0]], g0, gs0)
    pltpu.async_copy(s_hbm.at[rb.at[1]], g1, gs1)

    def body(k, c2):
      j0 = 2 * k
      j1 = j0 + 1
      jn0 = jnp.minimum(j0 + 2, IDXB - 1)
      jn1 = jnp.minimum(j0 + 3, IDXB - 1)
      pltpu.make_async_copy(s_hbm.at[rb.at[j0]], g0, gs0).wait()
      pltpu.async_copy(g0, acc.at[cb.at[j0]], ss0, add=True)
      pltpu.make_async_copy(s_hbm.at[rb.at[j1]], g1, gs1).wait()
      pltpu.async_copy(g1, acc.at[cb.at[j1]], ss1, add=True)
      pltpu.make_async_copy(g0, acc.at[cb.at[j0]], ss0).wait()
      pltpu.async_copy(s_hbm.at[rb.at[jn0]], g0, gs0)
      pltpu.make_async_copy(g1, acc.at[cb.at[j1]], ss1).wait()
      pltpu.async_copy(s_hbm.at[rb.at[jn1]], g1, gs1)
      return c2

    lax.fori_loop(0, IDXB // 2, body, 0)
    pltpu.make_async_copy(s_hbm.at[rb.at[IDXB - 1]], g0, gs0).wait()
    pltpu.make_async_copy(s_hbm.at[rb.at[IDXB - 1]], g1, gs1).wait()

  def prefetch(rb, cb, isem, blk):
    pltpu.async_copy(rows_hbm.at[pl.ds(cbase + blk * IDXB, IDXB)], rb, isem)
    pltpu.async_copy(cols_hbm.at[pl.ds(cbase + blk * IDXB, IDXB)], cb, isem)

  def hbody(h, carry):
    process(rowA, colA, isA)
    prefetch(rowA, colA, isA, jnp.minimum(2 * h + 2, nblk - 1))
    process(rowB, colB, isB)
    prefetch(rowB, colB, isB, jnp.minimum(2 * h + 3, nblk - 1))
    return carry

  lax.fori_loop(0, npairs, hbody, 0)
  pltpu.make_async_copy(rows_hbm.at[pl.ds(0, IDXB)], rowA, isA).wait()
  pltpu.make_async_copy(cols_hbm.at[pl.ds(0, IDXB)], colA, isA).wait()
  pltpu.make_async_copy(rows_hbm.at[pl.ds(0, IDXB)], rowB, isB).wait()
  pltpu.make_async_copy(cols_hbm.at[pl.ds(0, IDXB)], colB, isB).wait()
  plsc.subcore_barrier()
  _writeout(c, s, acc, out_hbm)


def _sc_scatter(s_val, rows3d, cols3d, zerosN):
  k = pl.kernel(
      _sc_scatter_body,
      out_type=jax.ShapeDtypeStruct((NC, NP, D), jnp.float32),
      mesh=_get_mesh(),
      scratch_types=[
          pltpu.VMEM((IDXB, CHUNK), jnp.int32),
          pltpu.VMEM((IDXB, CHUNK), jnp.int32),
          pltpu.VMEM((IDXB, CHUNK), jnp.int32),
          pltpu.VMEM((IDXB, CHUNK), jnp.int32),
          pltpu.VMEM((CHUNK, D), jnp.float32),
          pltpu.VMEM((CHUNK, D), jnp.float32),
          pltpu.SemaphoreType.DMA,
          pltpu.SemaphoreType.DMA,
          pltpu.SemaphoreType.DMA,
          pltpu.SemaphoreType.DMA,
          pltpu.SemaphoreType.DMA,
          pltpu.SemaphoreType.DMA,
          pltpu.VMEM_SHARED((NP, D), jnp.float32),
      ],
  )
  return k(s_val, rows3d, cols3d, zerosN)


def _sc_deg_body(ones_hbm, cols_hbm, zeros_hbm, out_hbm, colbuf, onesbuf, sem0,
                 acc):
  c = lax.axis_index("c")
  s = lax.axis_index("s")
  cbase = jnp.where(c == 0, s * NCH0, NS * NCH0 + s * NCH1)
  nch = jnp.where(c == 0, NCH0, NCH1)
  pltpu.sync_copy(ones_hbm.at[pl.ds(0, 112)], onesbuf)
  _init_acc(c, s, ones_hbm, zeros_hbm, acc)
  pltpu.sync_copy(cols_hbm.at[pl.ds(cbase, NCH0)], colbuf)
  plsc.subcore_barrier()

  def body(j, c2):
    pltpu.async_copy(onesbuf.at[pl.ds(0, CHUNK)], acc.at[colbuf.at[j]], sem0,
                     add=True)
    return c2

  lax.fori_loop(0, nch, body, 0)

  def drain(j, c2):
    pltpu.make_async_copy(onesbuf.at[pl.ds(0, CHUNK)], acc.at[colbuf.at[j]],
                          sem0).wait()
    return c2

  lax.fori_loop(0, nch, drain, 0)
  plsc.subcore_barrier()
  _writeout(c, s, acc, out_hbm)


def _sc_deg(onesN, cols3d, zerosN):
  k = pl.kernel(
      _sc_deg_body,
      out_type=jax.ShapeDtypeStruct((NC, NP, D), jnp.float32),
      mesh=_get_mesh(),
      scratch_types=[
          pltpu.VMEM((NCH0, CHUNK), jnp.int32),
          pltpu.VMEM((112, D), jnp.float32),
          pltpu.SemaphoreType.DMA,
          pltpu.VMEM_SHARED((NP, D), jnp.float32),
      ],
  )
  return k(onesN, cols3d, zerosN)


R = 1000
G = N // R

_DOT = dict(precision=lax.Precision.HIGHEST, preferred_element_type=jnp.float32)


def _ln(h, g, b):
  mu = jnp.mean(h, axis=-1, keepdims=True)
  d = h - mu
  var = jnp.mean(d * d, axis=-1, keepdims=True)
  return d * lax.rsqrt(var + 1e-5) * g + b


def _dinv(dd):
  return lax.rsqrt(dd[0, :, 0:1] + dd[1, :, 0:1])


def _gelu(x):
  return 0.5 * x * (1.0 + lax.erf(x * 0.7071067811865476))


def _tc1_body(x_ref, wp_ref, bp_ref, gp_ref, bep_ref, w1_ref, dd_ref,
              h0_ref, s1_ref):
  h = lax.dot_general(x_ref[...], wp_ref[...], (((1,), (1,)), ((), ())), **_DOT)
  h = _ln(h + bp_ref[...], gp_ref[...], bep_ref[...])
  h0_ref[...] = h
  s1_ref[...] = _dinv(dd_ref) * lax.dot_general(
      h, w1_ref[...], (((1,), (1,)), ((), ())), **_DOT)


def _tc_mid_body(h_ref, p_ref, dd_ref, b_ref, g_ref, be_ref, wn_ref,
                 hn_ref, sn_ref):
  dinv = _dinv(dd_ref)
  m = _gelu(dinv * (p_ref[0] + p_ref[1]) + b_ref[...])
  hn = _ln(h_ref[...] + m, g_ref[...], be_ref[...])
  hn_ref[...] = hn
  sn_ref[...] = dinv * lax.dot_general(
      hn, wn_ref[...], (((1,), (1,)), ((), ())), **_DOT)


def _tc_fin_body(h_ref, p_ref, dd_ref, b_ref, g_ref, be_ref, hn_ref):
  dinv = _dinv(dd_ref)
  m = _gelu(dinv * (p_ref[0] + p_ref[1]) + b_ref[...])
  hn_ref[...] = _ln(h_ref[...] + m, g_ref[...], be_ref[...])


_ROW = pl.BlockSpec((R, D), lambda i: (i, 0))
_W = pl.BlockSpec((D, D), lambda i: (0, 0))
_VEC = pl.BlockSpec((1, D), lambda i: (0, 0))
_DD = pl.BlockSpec((NC, R, D), lambda i: (0, i, 0))
_P = pl.BlockSpec((NC, R, D), lambda i: (0, i, 0))
_OUT2 = [jax.ShapeDtypeStruct((N, D), jnp.float32)] * 2
_OUT1 = jax.ShapeDtypeStruct((N, D), jnp.float32)


def _tc1(x, wp, bp, gp, bep, w1, dd):
  return pl.pallas_call(
      _tc1_body, grid=(G,),
      in_specs=[_ROW, _W, _VEC, _VEC, _VEC, _W, _DD],
      out_specs=[_ROW, _ROW], out_shape=_OUT2,
  )(x, wp, bp, gp, bep, w1, dd)


def _tc_mid(h, p, dd, b, g, be, wn):
  return pl.pallas_call(
      _tc_mid_body, grid=(G,),
      in_specs=[_ROW, _P, _DD, _VEC, _VEC, _VEC, _W],
      out_specs=[_ROW, _ROW], out_shape=_OUT2,
  )(h, p, dd, b, g, be, wn)


def _tc_fin(h, p, dd, b, g, be):
  return pl.pallas_call(
      _tc_fin_body, grid=(G,),
      in_specs=[_ROW, _P, _DD, _VEC, _VEC, _VEC],
      out_specs=_ROW, out_shape=_OUT1,
  )(h, p, dd, b, g, be)


def kernel(x, edge_index, W_proj, b_proj, g_proj, be_proj,
           W1, b1, g1, be1, W2, b2, g2, be2, W3, b3, g3, be3):
  pad = EPAD - E
  rows = jnp.concatenate([edge_index[0], jnp.zeros((pad,), jnp.int32)])
  cols = jnp.concatenate([edge_index[1], jnp.full((pad,), N, jnp.int32)])
  rows3d = rows.reshape(TOTC, CHUNK)
  cols3d = cols.reshape(TOTC, CHUNK)
  zerosN = jnp.zeros((NP, D), jnp.float32)
  onesN = jnp.ones((N, D), jnp.float32)

  dd = _sc_deg(onesN, cols3d, zerosN)

  r2 = lambda v: v.reshape(1, D)
  h0, s1 = _tc1(x, W_proj, r2(b_proj), r2(g_proj), r2(be_proj), W1, dd)
  p1 = _sc_scatter(s1, rows3d, cols3d, zerosN)
  h1, s2 = _tc_mid(h0, p1, dd, r2(b1), r2(g1), r2(be1), W2)
  p2 = _sc_scatter(s2, rows3d, cols3d, zerosN)
  h2, s3 = _tc_mid(h1, p2, dd, r2(b2), r2(g2), r2(be2), W3)
  p3 = _sc_scatter(s3, rows3d, cols3d, zerosN)
  return _tc_fin(h2, p3, dd, r2(b3), r2(g3), r2(be3))


# speedup vs baseline: 2.0493x; 1.0641x over previous
import functools

import jax
import jax.numpy as jnp
from jax import lax
from jax.experimental import pallas as pl
from jax.experimental.pallas import tpu as pltpu
from jax.experimental.pallas import tpu_sc as plsc

N = 10000
E = 320000
D = 128

NC = 2
NS = 16
NW = NC * NS
CHUNK = 105
IDXB = 24
NCH0 = 144
NCH1 = 48
TOTC = NS * (NCH0 + NCH1) + 96
NPROC = NS * (NCH0 + NCH1)
EPAD = TOTC * CHUNK
NP = 10008
RPT = 632
RL_OUT = NP - 15 * RPT
RL_S = N - 15 * RPT

@functools.cache
def _get_mesh():
  return plsc.VectorSubcoreMesh(core_axis_name="c", subcore_axis_name="s",
                                num_cores=NC, num_subcores=NS)


def _init_acc(c, s, init_hbm, zeros_hbm, acc):
  last = s == NS - 1

  @pl.when(jnp.logical_and(c == 0, jnp.logical_not(last)))
  def _():
    pltpu.sync_copy(init_hbm.at[pl.ds(s * RPT, RPT)],
                    acc.at[pl.ds(s * RPT, RPT)])

  @pl.when(jnp.logical_and(c == 0, last))
  def _():
    pltpu.sync_copy(init_hbm.at[pl.ds(15 * RPT, RL_S)],
                    acc.at[pl.ds(15 * RPT, RL_S)])

  @pl.when(jnp.logical_and(c == 1, jnp.logical_not(last)))
  def _():
    pltpu.sync_copy(zeros_hbm.at[pl.ds(s * RPT, RPT)],
                    acc.at[pl.ds(s * RPT, RPT)])

  @pl.when(jnp.logical_and(c == 1, last))
  def _():
    pltpu.sync_copy(zeros_hbm.at[pl.ds(15 * RPT, RL_OUT)],
                    acc.at[pl.ds(15 * RPT, RL_OUT)])


def _writeout(c, s, acc, out_hbm):
  last = s == NS - 1

  @pl.when(jnp.logical_not(last))
  def _():
    pltpu.sync_copy(acc.at[pl.ds(s * RPT, RPT)],
                    out_hbm.at[c, pl.ds(s * RPT, RPT)])

  @pl.when(last)
  def _():
    pltpu.sync_copy(acc.at[pl.ds(15 * RPT, RL_OUT)],
                    out_hbm.at[c, pl.ds(15 * RPT, RL_OUT)])


def _sc_scatter_body(s_hbm, rows_hbm, cols_hbm, zeros_hbm, out_hbm,
                     rowA, colA, rowB, colB, g0, g1,
                     isA, isB, gs0, gs1, ss0, ss1, acc):
  c = lax.axis_index("c")
  s = lax.axis_index("s")
  cbase = jnp.where(c == 0, s * NCH0, NS * NCH0 + s * NCH1)
  nblk = jnp.where(c == 0, NCH0 // IDXB, NCH1 // IDXB)
  npairs = jnp.where(c == 0, NCH0 // (2 * IDXB), NCH1 // (2 * IDXB))
  pltpu.async_copy(rows_hbm.at[pl.ds(cbase, IDXB)], rowA, isA)
  pltpu.async_copy(cols_hbm.at[pl.ds(cbase, IDXB)], colA, isA)
  pltpu.async_copy(rows_hbm.at[pl.ds(cbase + IDXB, IDXB)], rowB, isB)
  pltpu.async_copy(cols_hbm.at[pl.ds(cbase + IDXB, IDXB)], colB, isB)
  _init_acc(c, s, s_hbm, zeros_hbm, acc)
  plsc.subcore_barrier()

  def process(rb, cb, isem):
    pltpu.make_async_copy(rows_hbm.at[pl.ds(0, IDXB)], rb, isem).wait()
    pltpu.make_async_copy(cols_hbm.at[pl.ds(0, IDXB)], cb, isem).wait()
    pltpu.async_copy(s_hbm.at[rb.at[0]], g0, gs0)
    pltpu.async_copy(s_hbm.at[rb.at[1]], g1, gs1)

    def body(k, c2):
      j0 = 2 * k
      j1 = j0 + 1
      jn0 = jnp.minimum(j0 + 2, IDXB - 1)
      jn1 = jnp.minimum(j0 + 3, IDXB - 1)
      pltpu.make_async_copy(s_hbm.at[rb.at[j0]], g0, gs0).wait()
      pltpu.async_copy(g0, acc.at[cb.at[j0]], ss0, add=True)
      pltpu.make_async_copy(s_hbm.at[rb.at[j1]], g1, gs1).wait()
      pltpu.async_copy(g1, acc.at[cb.at[j1]], ss1, add=True)
      pltpu.make_async_copy(g0, acc.at[cb.at[j0]], ss0).wait()
      pltpu.async_copy(s_hbm.at[rb.at[jn0]], g0, gs0)
      pltpu.make_async_copy(g1, acc.at[cb.at[j1]], ss1).wait()
      pltpu.async_copy(s_hbm.at[rb.at[jn1]], g1, gs1)
      return c2

    lax.fori_loop(0, IDXB // 2, body, 0)
    pltpu.make_async_copy(s_hbm.at[rb.at[IDXB - 1]], g0, gs0).wait()
    pltpu.make_async_copy(s_hbm.at[rb.at[IDXB - 1]], g1, gs1).wait()

  def prefetch(rb, cb, isem, blk):
    pltpu.async_copy(rows_hbm.at[pl.ds(cbase + blk * IDXB, IDXB)], rb, isem)
    pltpu.async_copy(cols_hbm.at[pl.ds(cbase + blk * IDXB, IDXB)], cb, isem)

  def hbody(h, carry):
    process(rowA, colA, isA)
    prefetch(rowA, colA, isA, jnp.minimum(2 * h + 2, nblk - 1))
    process(rowB, colB, isB)
    prefetch(rowB, colB, isB, jnp.minimum(2 * h + 3, nblk - 1))
    return carry

  lax.fori_loop(0, npairs, hbody, 0)
  pltpu.make_async_copy(rows_hbm.at[pl.ds(0, IDXB)], rowA, isA).wait()
  pltpu.make_async_copy(cols_hbm.at[pl.ds(0, IDXB)], colA, isA).wait()
  pltpu.make_async_copy(rows_hbm.at[pl.ds(0, IDXB)], rowB, isB).wait()
  pltpu.make_async_copy(cols_hbm.at[pl.ds(0, IDXB)], colB, isB).wait()
  plsc.subcore_barrier()
  _writeout(c, s, acc, out_hbm)


def _sc_scatter(s_val, rows3d, cols3d, zerosN):
  k = pl.kernel(
      _sc_scatter_body,
      out_type=jax.ShapeDtypeStruct((NC, NP, D), jnp.float32),
      mesh=_get_mesh(),
      scratch_types=[
          pltpu.VMEM((IDXB, CHUNK), jnp.int32),
          pltpu.VMEM((IDXB, CHUNK), jnp.int32),
          pltpu.VMEM((IDXB, CHUNK), jnp.int32),
          pltpu.VMEM((IDXB, CHUNK), jnp.int32),
          pltpu.VMEM((CHUNK, D), jnp.float32),
          pltpu.VMEM((CHUNK, D), jnp.float32),
          pltpu.SemaphoreType.DMA,
          pltpu.SemaphoreType.DMA,
          pltpu.SemaphoreType.DMA,
          pltpu.SemaphoreType.DMA,
          pltpu.SemaphoreType.DMA,
          pltpu.SemaphoreType.DMA,
          pltpu.VMEM_SHARED((NP, D), jnp.float32),
      ],
  )
  return k(s_val, rows3d, cols3d, zerosN)


def _sc_deg_body(ones_hbm, cols_hbm, zeros_hbm, out_hbm, colbuf, onesbuf, sem0,
                 acc):
  c = lax.axis_index("c")
  s = lax.axis_index("s")
  wid = c * NS + s
  pltpu.sync_copy(ones_hbm.at[pl.ds(0, 112)], onesbuf)
  _init_acc(c, s, ones_hbm, zeros_hbm, acc)
  pltpu.sync_copy(cols_hbm.at[pl.ds(wid * (NPROC // NW), NPROC // NW)], colbuf)
  plsc.subcore_barrier()

  def body(j, c2):
    pltpu.async_copy(onesbuf.at[pl.ds(0, CHUNK)], acc.at[colbuf.at[j]], sem0,
                     add=True)
    return c2

  lax.fori_loop(0, NPROC // NW, body, 0)

  def drain(j, c2):
    pltpu.make_async_copy(onesbuf.at[pl.ds(0, CHUNK)], acc.at[colbuf.at[j]],
                          sem0).wait()
    return c2

  lax.fori_loop(0, NPROC // NW, drain, 0)
  plsc.subcore_barrier()
  _writeout(c, s, acc, out_hbm)


def _sc_deg(onesN, cols3d, zerosN):
  k = pl.kernel(
      _sc_deg_body,
      out_type=jax.ShapeDtypeStruct((NC, NP, D), jnp.float32),
      mesh=_get_mesh(),
      scratch_types=[
          pltpu.VMEM((NPROC // NW, CHUNK), jnp.int32),
          pltpu.VMEM((112, D), jnp.float32),
          pltpu.SemaphoreType.DMA,
          pltpu.VMEM_SHARED((NP, D), jnp.float32),
      ],
  )
  return k(onesN, cols3d, zerosN)


R = 1000
G = N // R

_DOT = dict(precision=lax.Precision.DEFAULT, preferred_element_type=jnp.float32)


def _ln(h, g, b):
  mu = jnp.mean(h, axis=-1, keepdims=True)
  d = h - mu
  var = jnp.mean(d * d, axis=-1, keepdims=True)
  return d * lax.rsqrt(var + 1e-5) * g + b


def _dinv(dd):
  return lax.rsqrt(dd[0, :, 0:1] + dd[1, :, 0:1])


def _gelu(x):
  return 0.5 * x * (1.0 + lax.erf(x * 0.7071067811865476))


def _tc1_body(x_ref, wp_ref, bp_ref, gp_ref, bep_ref, w1_ref, dd_ref,
              h0_ref, s1_ref):
  h = lax.dot_general(x_ref[...], wp_ref[...], (((1,), (1,)), ((), ())), **_DOT)
  h = _ln(h + bp_ref[...], gp_ref[...], bep_ref[...])
  h0_ref[...] = h
  s1_ref[...] = _dinv(dd_ref) * lax.dot_general(
      h, w1_ref[...], (((1,), (1,)), ((), ())), **_DOT)


def _tc_mid_body(h_ref, p_ref, dd_ref, b_ref, g_ref, be_ref, wn_ref,
                 hn_ref, sn_ref):
  dinv = _dinv(dd_ref)
  m = _gelu(dinv * (p_ref[0] + p_ref[1]) + b_ref[...])
  hn = _ln(h_ref[...] + m, g_ref[...], be_ref[...])
  hn_ref[...] = hn
  sn_ref[...] = dinv * lax.dot_general(
      hn, wn_ref[...], (((1,), (1,)), ((), ())), **_DOT)


def _tc_fin_body(h_ref, p_ref, dd_ref, b_ref, g_ref, be_ref, hn_ref):
  dinv = _dinv(dd_ref)
  m = _gelu(dinv * (p_ref[0] + p_ref[1]) + b_ref[...])
  hn_ref[...] = _ln(h_ref[...] + m, g_ref[...], be_ref[...])


_ROW = pl.BlockSpec((R, D), lambda i: (i, 0))
_W = pl.BlockSpec((D, D), lambda i: (0, 0))
_VEC = pl.BlockSpec((1, D), lambda i: (0, 0))
_DD = pl.BlockSpec((NC, R, D), lambda i: (0, i, 0))
_P = pl.BlockSpec((NC, R, D), lambda i: (0, i, 0))
_OUT2 = [jax.ShapeDtypeStruct((N, D), jnp.float32)] * 2
_OUT1 = jax.ShapeDtypeStruct((N, D), jnp.float32)


def _tc1(x, wp, bp, gp, bep, w1, dd):
  return pl.pallas_call(
      _tc1_body, grid=(G,),
      in_specs=[_ROW, _W, _VEC, _VEC, _VEC, _W, _DD],
      out_specs=[_ROW, _ROW], out_shape=_OUT2,
  )(x, wp, bp, gp, bep, w1, dd)


def _tc_mid(h, p, dd, b, g, be, wn):
  return pl.pallas_call(
      _tc_mid_body, grid=(G,),
      in_specs=[_ROW, _P, _DD, _VEC, _VEC, _VEC, _W],
      out_specs=[_ROW, _ROW], out_shape=_OUT2,
  )(h, p, dd, b, g, be, wn)


def _tc_fin(h, p, dd, b, g, be):
  return pl.pallas_call(
      _tc_fin_body, grid=(G,),
      in_specs=[_ROW, _P, _DD, _VEC, _VEC, _VEC],
      out_specs=_ROW, out_shape=_OUT1,
  )(h, p, dd, b, g, be)


def kernel(x, edge_index, W_proj, b_proj, g_proj, be_proj,
           W1, b1, g1, be1, W2, b2, g2, be2, W3, b3, g3, be3):
  pad = EPAD - E
  rows = jnp.concatenate([edge_index[0], jnp.zeros((pad,), jnp.int32)])
  cols = jnp.concatenate([edge_index[1], jnp.full((pad,), N, jnp.int32)])
  rows3d = rows.reshape(TOTC, CHUNK)
  cols3d = cols.reshape(TOTC, CHUNK)
  zerosN = jnp.zeros((NP, D), jnp.float32)
  onesN = jnp.ones((N, D), jnp.float32)

  dd = _sc_deg(onesN, cols3d, zerosN)

  r2 = lambda v: v.reshape(1, D)
  h0, s1 = _tc1(x, W_proj, r2(b_proj), r2(g_proj), r2(be_proj), W1, dd)
  p1 = _sc_scatter(s1, rows3d, cols3d, zerosN)
  h1, s2 = _tc_mid(h0, p1, dd, r2(b1), r2(g1), r2(be1), W2)
  p2 = _sc_scatter(s2, rows3d, cols3d, zerosN)
  h2, s3 = _tc_mid(h1, p2, dd, r2(b2), r2(g2), r2(be2), W3)
  p3 = _sc_scatter(s3, rows3d, cols3d, zerosN)
  return _tc_fin(h2, p3, dd, r2(b3), r2(g3), r2(be3))


# speedup vs baseline: 2.0720x; 1.0111x over previous
import functools

import jax
import jax.numpy as jnp
from jax import lax
from jax.experimental import pallas as pl
from jax.experimental.pallas import tpu as pltpu
from jax.experimental.pallas import tpu_sc as plsc

N = 10000
E = 320000
D = 128

NC = 2
NS = 16
NW = NC * NS
CHUNK = 105
IDXB = 24
NCH0 = 144
NCH1 = 48
TOTC = NS * (NCH0 + NCH1) + 96
NPROC = NS * (NCH0 + NCH1)
EPAD = TOTC * CHUNK
NP = 10008
RPT = 632
RL_OUT = NP - 15 * RPT
RL_S = N - 15 * RPT

@functools.cache
def _get_mesh():
  return plsc.VectorSubcoreMesh(core_axis_name="c", subcore_axis_name="s",
                                num_cores=NC, num_subcores=NS)


def _init_acc(c, s, init_hbm, zeros_hbm, acc):
  last = s == NS - 1

  @pl.when(jnp.logical_and(c == 0, jnp.logical_not(last)))
  def _():
    pltpu.sync_copy(init_hbm.at[pl.ds(s * RPT, RPT)],
                    acc.at[pl.ds(s * RPT, RPT)])

  @pl.when(jnp.logical_and(c == 0, last))
  def _():
    pltpu.sync_copy(init_hbm.at[pl.ds(15 * RPT, RL_S)],
                    acc.at[pl.ds(15 * RPT, RL_S)])

  @pl.when(jnp.logical_and(c == 1, jnp.logical_not(last)))
  def _():
    pltpu.sync_copy(zeros_hbm.at[pl.ds(s * RPT, RPT)],
                    acc.at[pl.ds(s * RPT, RPT)])

  @pl.when(jnp.logical_and(c == 1, last))
  def _():
    pltpu.sync_copy(zeros_hbm.at[pl.ds(15 * RPT, RL_OUT)],
                    acc.at[pl.ds(15 * RPT, RL_OUT)])


def _writeout(c, s, acc, out_hbm):
  last = s == NS - 1

  @pl.when(jnp.logical_not(last))
  def _():
    pltpu.sync_copy(acc.at[pl.ds(s * RPT, RPT)],
                    out_hbm.at[c, pl.ds(s * RPT, RPT)])

  @pl.when(last)
  def _():
    pltpu.sync_copy(acc.at[pl.ds(15 * RPT, RL_OUT)],
                    out_hbm.at[c, pl.ds(15 * RPT, RL_OUT)])


def _sc_scatter_body(s_hbm, rows_hbm, cols_hbm, zeros_hbm, out_hbm,
                     rowA, colA, rowB, colB, g0, g1,
                     isA, isB, gs0, gs1, ss0, ss1, acc):
  c = lax.axis_index("c")
  s = lax.axis_index("s")
  cbase = jnp.where(c == 0, s * NCH0, NS * NCH0 + s * NCH1)
  nblk = jnp.where(c == 0, NCH0 // IDXB, NCH1 // IDXB)
  npairs = jnp.where(c == 0, NCH0 // (2 * IDXB), NCH1 // (2 * IDXB))
  pltpu.async_copy(rows_hbm.at[pl.ds(cbase, IDXB)], rowA, isA)
  pltpu.async_copy(cols_hbm.at[pl.ds(cbase, IDXB)], colA, isA)
  pltpu.async_copy(rows_hbm.at[pl.ds(cbase + IDXB, IDXB)], rowB, isB)
  pltpu.async_copy(cols_hbm.at[pl.ds(cbase + IDXB, IDXB)], colB, isB)
  _init_acc(c, s, s_hbm, zeros_hbm, acc)
  plsc.subcore_barrier()

  def process(rb, cb, isem):
    pltpu.make_async_copy(rows_hbm.at[pl.ds(0, IDXB)], rb, isem).wait()
    pltpu.make_async_copy(cols_hbm.at[pl.ds(0, IDXB)], cb, isem).wait()
    pltpu.async_copy(s_hbm.at[rb.at[0]], g0, gs0)
    pltpu.async_copy(s_hbm.at[rb.at[1]], g1, gs1)

    def body(k, c2):
      j0 = 2 * k
      j1 = j0 + 1
      jn0 = jnp.minimum(j0 + 2, IDXB - 1)
      jn1 = jnp.minimum(j0 + 3, IDXB - 1)
      pltpu.make_async_copy(s_hbm.at[rb.at[j0]], g0, gs0).wait()
      pltpu.async_copy(g0, acc.at[cb.at[j0]], ss0, add=True)
      pltpu.make_async_copy(s_hbm.at[rb.at[j1]], g1, gs1).wait()
      pltpu.async_copy(g1, acc.at[cb.at[j1]], ss1, add=True)
      pltpu.make_async_copy(g0, acc.at[cb.at[j0]], ss0).wait()
      pltpu.async_copy(s_hbm.at[rb.at[jn0]], g0, gs0)
      pltpu.make_async_copy(g1, acc.at[cb.at[j1]], ss1).wait()
      pltpu.async_copy(s_hbm.at[rb.at[jn1]], g1, gs1)
      return c2

    lax.fori_loop(0, IDXB // 2, body, 0)
    pltpu.make_async_copy(s_hbm.at[rb.at[IDXB - 1]], g0, gs0).wait()
    pltpu.make_async_copy(s_hbm.at[rb.at[IDXB - 1]], g1, gs1).wait()

  def prefetch(rb, cb, isem, blk):
    pltpu.async_copy(rows_hbm.at[pl.ds(cbase + blk * IDXB, IDXB)], rb, isem)
    pltpu.async_copy(cols_hbm.at[pl.ds(cbase + blk * IDXB, IDXB)], cb, isem)

  def hbody(h, carry):
    process(rowA, colA, isA)
    prefetch(rowA, colA, isA, jnp.minimum(2 * h + 2, nblk - 1))
    process(rowB, colB, isB)
    prefetch(rowB, colB, isB, jnp.minimum(2 * h + 3, nblk - 1))
    return carry

  lax.fori_loop(0, npairs, hbody, 0)
  pltpu.make_async_copy(rows_hbm.at[pl.ds(0, IDXB)], rowA, isA).wait()
  pltpu.make_async_copy(cols_hbm.at[pl.ds(0, IDXB)], colA, isA).wait()
  pltpu.make_async_copy(rows_hbm.at[pl.ds(0, IDXB)], rowB, isB).wait()
  pltpu.make_async_copy(cols_hbm.at[pl.ds(0, IDXB)], colB, isB).wait()
  plsc.subcore_barrier()
  _writeout(c, s, acc, out_hbm)


def _sc_scatter(s_val, rows3d, cols3d, zerosN):
  k = pl.kernel(
      _sc_scatter_body,
      out_type=jax.ShapeDtypeStruct((NC, NP, D), jnp.float32),
      mesh=_get_mesh(),
      scratch_types=[
          pltpu.VMEM((IDXB, CHUNK), jnp.int32),
          pltpu.VMEM((IDXB, CHUNK), jnp.int32),
          pltpu.VMEM((IDXB, CHUNK), jnp.int32),
          pltpu.VMEM((IDXB, CHUNK), jnp.int32),
          pltpu.VMEM((CHUNK, D), jnp.float32),
          pltpu.VMEM((CHUNK, D), jnp.float32),
          pltpu.SemaphoreType.DMA,
          pltpu.SemaphoreType.DMA,
          pltpu.SemaphoreType.DMA,
          pltpu.SemaphoreType.DMA,
          pltpu.SemaphoreType.DMA,
          pltpu.SemaphoreType.DMA,
          pltpu.VMEM_SHARED((NP, D), jnp.float32),
      ],
  )
  return k(s_val, rows3d, cols3d, zerosN)


def _sc_deg_body(ones_hbm, cols_hbm, zeros_hbm, out_hbm, colbuf, onesbuf, sem0,
                 acc):
  c = lax.axis_index("c")
  s = lax.axis_index("s")
  wid = c * NS + s
  pltpu.sync_copy(ones_hbm.at[pl.ds(0, 112)], onesbuf)
  _init_acc(c, s, ones_hbm, zeros_hbm, acc)
  pltpu.sync_copy(cols_hbm.at[pl.ds(wid * (NPROC // NW), NPROC // NW)], colbuf)
  plsc.subcore_barrier()

  def body(j, c2):
    pltpu.async_copy(onesbuf.at[pl.ds(0, CHUNK)], acc.at[colbuf.at[j]], sem0,
                     add=True)
    return c2

  lax.fori_loop(0, NPROC // NW, body, 0)

  def drain(j, c2):
    pltpu.make_async_copy(onesbuf.at[pl.ds(0, CHUNK)], acc.at[colbuf.at[j]],
                          sem0).wait()
    return c2

  lax.fori_loop(0, NPROC // NW, drain, 0)
  plsc.subcore_barrier()
  _writeout(c, s, acc, out_hbm)


def _sc_deg(onesN, cols3d, zerosN):
  k = pl.kernel(
      _sc_deg_body,
      out_type=jax.ShapeDtypeStruct((NC, NP, D), jnp.float32),
      mesh=_get_mesh(),
      scratch_types=[
          pltpu.VMEM((NPROC // NW, CHUNK), jnp.int32),
          pltpu.VMEM((112, D), jnp.float32),
          pltpu.SemaphoreType.DMA,
          pltpu.VMEM_SHARED((NP, D), jnp.float32),
      ],
  )
  return k(onesN, cols3d, zerosN)


R = 2000
G = N // R

_DOT = dict(precision=lax.Precision.DEFAULT, preferred_element_type=jnp.float32)


def _ln(h, g, b):
  mu = jnp.mean(h, axis=-1, keepdims=True)
  d = h - mu
  var = jnp.mean(d * d, axis=-1, keepdims=True)
  return d * lax.rsqrt(var + 1e-5) * g + b


def _dinv(dd):
  return lax.rsqrt(dd[0, :, 0:1] + dd[1, :, 0:1])


def _gelu(x):
  return 0.5 * x * (1.0 + lax.erf(x * 0.7071067811865476))


def _tc1_body(x_ref, wp_ref, bp_ref, gp_ref, bep_ref, w1_ref, dd_ref,
              h0_ref, s1_ref):
  h = lax.dot_general(x_ref[...], wp_ref[...], (((1,), (1,)), ((), ())), **_DOT)
  h = _ln(h + bp_ref[...], gp_ref[...], bep_ref[...])
  h0_ref[...] = h
  s1_ref[...] = _dinv(dd_ref) * lax.dot_general(
      h, w1_ref[...], (((1,), (1,)), ((), ())), **_DOT)


def _tc_mid_body(h_ref, p_ref, dd_ref, b_ref, g_ref, be_ref, wn_ref,
                 hn_ref, sn_ref):
  dinv = _dinv(dd_ref)
  m = _gelu(dinv * (p_ref[0] + p_ref[1]) + b_ref[...])
  hn = _ln(h_ref[...] + m, g_ref[...], be_ref[...])
  hn_ref[...] = hn
  sn_ref[...] = dinv * lax.dot_general(
      hn, wn_ref[...], (((1,), (1,)), ((), ())), **_DOT)


def _tc_fin_body(h_ref, p_ref, dd_ref, b_ref, g_ref, be_ref, hn_ref):
  dinv = _dinv(dd_ref)
  m = _gelu(dinv * (p_ref[0] + p_ref[1]) + b_ref[...])
  hn_ref[...] = _ln(h_ref[...] + m, g_ref[...], be_ref[...])


_ROW = pl.BlockSpec((R, D), lambda i: (i, 0))
_W = pl.BlockSpec((D, D), lambda i: (0, 0))
_VEC = pl.BlockSpec((1, D), lambda i: (0, 0))
_DD = pl.BlockSpec((NC, R, D), lambda i: (0, i, 0))
_P = pl.BlockSpec((NC, R, D), lambda i: (0, i, 0))
_OUT2 = [jax.ShapeDtypeStruct((N, D), jnp.float32)] * 2
_OUT1 = jax.ShapeDtypeStruct((N, D), jnp.float32)


def _tc1(x, wp, bp, gp, bep, w1, dd):
  return pl.pallas_call(
      _tc1_body, grid=(G,),
      in_specs=[_ROW, _W, _VEC, _VEC, _VEC, _W, _DD],
      out_specs=[_ROW, _ROW], out_shape=_OUT2,
  )(x, wp, bp, gp, bep, w1, dd)


def _tc_mid(h, p, dd, b, g, be, wn):
  return pl.pallas_call(
      _tc_mid_body, grid=(G,),
      in_specs=[_ROW, _P, _DD, _VEC, _VEC, _VEC, _W],
      out_specs=[_ROW, _ROW], out_shape=_OUT2,
  )(h, p, dd, b, g, be, wn)


def _tc_fin(h, p, dd, b, g, be):
  return pl.pallas_call(
      _tc_fin_body, grid=(G,),
      in_specs=[_ROW, _P, _DD, _VEC, _VEC, _VEC],
      out_specs=_ROW, out_shape=_OUT1,
  )(h, p, dd, b, g, be)


def kernel(x, edge_index, W_proj, b_proj, g_proj, be_proj,
           W1, b1, g1, be1, W2, b2, g2, be2, W3, b3, g3, be3):
  pad = EPAD - E
  rows = jnp.concatenate([edge_index[0], jnp.zeros((pad,), jnp.int32)])
  cols = jnp.concatenate([edge_index[1], jnp.full((pad,), N, jnp.int32)])
  rows3d = rows.reshape(TOTC, CHUNK)
  cols3d = cols.reshape(TOTC, CHUNK)
  zerosN = jnp.zeros((NP, D), jnp.float32)
  onesN = jnp.ones((N, D), jnp.float32)

  dd = _sc_deg(onesN, cols3d, zerosN)

  r2 = lambda v: v.reshape(1, D)
  h0, s1 = _tc1(x, W_proj, r2(b_proj), r2(g_proj), r2(be_proj), W1, dd)
  p1 = _sc_scatter(s1, rows3d, cols3d, zerosN)
  h1, s2 = _tc_mid(h0, p1, dd, r2(b1), r2(g1), r2(be1), W2)
  p2 = _sc_scatter(s2, rows3d, cols3d, zerosN)
  h2, s3 = _tc_mid(h1, p2, dd, r2(b2), r2(g2), r2(be2), W3)
  p3 = _sc_scatter(s3, rows3d, cols3d, zerosN)
  return _tc_fin(h2, p3, dd, r2(b3), r2(g3), r2(be3))
